# Initial kernel scaffold; baseline (speedup 1.0000x reference)
import functools

import jax
import jax.numpy as jnp
from jax import lax
from jax.experimental import pallas as pl
from jax.experimental.pallas import tpu as pltpu
from jax.experimental.pallas import tpu_sc as plsc

N_NODES = 10000
N_EDGES = 320000
D = 128

NC = 2
NS = 16
NW = NC * NS
CH = 128
NCHUNK = N_EDGES // CH
JMAX = (NCHUNK + NW - 1) // NW
NPAD = 10240
ROWS_PT = N_NODES // NS
CNT_PT = NPAD // NS

_MESH = plsc.VectorSubcoreMesh(core_axis_name="c", subcore_axis_name="s")


def _sc1_body(y_h, src_h, dst_h, zrow_h, zcnt_h,
              sums_h, cnt_h,
              acc_s, cnt_s,
              sidx_v, didx_v, rows_v, ones_v, sem):
    cid = lax.axis_index("c")
    sid = lax.axis_index("s")
    wid = sid * NC + cid

    for k in range(CH // 16):
        ones_v[pl.ds(k * 16, 16)] = jnp.ones((16,), jnp.float32)

    pltpu.sync_copy(zrow_h, acc_s.at[pl.ds(sid * ROWS_PT, ROWS_PT)])
    pltpu.sync_copy(zcnt_h, cnt_s.at[pl.ds(sid * CNT_PT, CNT_PT)])
    plsc.subcore_barrier()

    def body(j, carry):
        c = j * NW + wid

        @pl.when(c < NCHUNK)
        def _():
            base = c * CH
            pltpu.sync_copy(src_h.at[pl.ds(base, CH)], sidx_v)
            pltpu.async_copy(y_h.at[sidx_v], rows_v, sem).wait()
            pltpu.sync_copy(dst_h.at[pl.ds(base, CH)], didx_v)
            pltpu.sync_copy(rows_v, acc_s.at[didx_v], add=True)
            pltpu.sync_copy(ones_v, cnt_s.at[didx_v], add=True)

        return carry

    lax.fori_loop(0, JMAX, body, 0)
    plsc.subcore_barrier()

    pltpu.sync_copy(
        acc_s.at[pl.ds(sid * ROWS_PT, ROWS_PT)],
        sums_h.at[pl.ds(cid * N_NODES + sid * ROWS_PT, ROWS_PT)])
    pltpu.sync_copy(
        cnt_s.at[pl.ds(sid * CNT_PT, CNT_PT)],
        cnt_h.at[pl.ds(cid * NPAD + sid * CNT_PT, CNT_PT)])


_sc1 = functools.partial(
    pl.kernel,
    mesh=_MESH,
    out_type=[
        jax.ShapeDtypeStruct((NC * N_NODES, D), jnp.float32),
        jax.ShapeDtypeStruct((NC * NPAD,), jnp.float32),
    ],
    scratch_types=[
        pltpu.VMEM_SHARED((N_NODES, D), jnp.float32),
        pltpu.VMEM_SHARED((NPAD,), jnp.float32),
        pltpu.VMEM((CH,), jnp.int32),
        pltpu.VMEM((CH,), jnp.int32),
        pltpu.VMEM((CH, D), jnp.float32),
        pltpu.VMEM((CH,), jnp.float32),
        pltpu.SemaphoreType.DMA,
    ],
)(_sc1_body)


def _sc2_body(s_h, src_h, dst_h, zcnt_h,
              out_h,
              acc_s,
              s_v, sidx_v, didx_v, vals_v):
    cid = lax.axis_index("c")
    sid = lax.axis_index("s")
    wid = sid * NC + cid

    pltpu.sync_copy(s_h, s_v)
    pltpu.sync_copy(zcnt_h, acc_s.at[pl.ds(sid * CNT_PT, CNT_PT)])
    plsc.subcore_barrier()

    def body(j, carry):
        c = j * NW + wid

        @pl.when(c < NCHUNK)
        def _():
            base = c * CH
            pltpu.sync_copy(src_h.at[pl.ds(base, CH)], sidx_v)
            for k in range(CH // 16):
                idx16 = sidx_v[pl.ds(k * 16, 16)]
                vals_v[pl.ds(k * 16, 16)] = plsc.load_gather(s_v, [idx16])
            pltpu.sync_copy(dst_h.at[pl.ds(base, CH)], didx_v)
            pltpu.sync_copy(vals_v, acc_s.at[didx_v], add=True)

        return carry

    lax.fori_loop(0, JMAX, body, 0)
    plsc.subcore_barrier()

    pltpu.sync_copy(
        acc_s.at[pl.ds(sid * CNT_PT, CNT_PT)],
        out_h.at[pl.ds(cid * NPAD + sid * CNT_PT, CNT_PT)])


_sc2 = functools.partial(
    pl.kernel,
    mesh=_MESH,
    out_type=[jax.ShapeDtypeStruct((NC * NPAD,), jnp.float32)],
    scratch_types=[
        pltpu.VMEM_SHARED((NPAD,), jnp.float32),
        pltpu.VMEM((NPAD,), jnp.float32),
        pltpu.VMEM((CH,), jnp.int32),
        pltpu.VMEM((CH,), jnp.int32),
        pltpu.VMEM((CH,), jnp.float32),
    ],
)(_sc2_body)


_RB = 1000


def _tc1_body(x_ref, wl_ref, wr_ref, b1_ref, y_ref, z_ref):
    xb = x_ref[...]
    y_ref[...] = jnp.dot(xb, wl_ref[...], preferred_element_type=jnp.float32)
    z_ref[...] = (jnp.dot(xb, wr_ref[...], preferred_element_type=jnp.float32)
                  + b1_ref[...])


def _tc1(x, wlT, wrT, b1):
    return pl.pallas_call(
        _tc1_body,
        grid=(N_NODES // _RB,),
        in_specs=[
            pl.BlockSpec((_RB, D), lambda i: (i, 0)),
            pl.BlockSpec((D, D), lambda i: (0, 0)),
            pl.BlockSpec((D, D), lambda i: (0, 0)),
            pl.BlockSpec((1, D), lambda i: (0, 0)),
        ],
        out_specs=[
            pl.BlockSpec((_RB, D), lambda i: (i, 0)),
            pl.BlockSpec((_RB, D), lambda i: (i, 0)),
        ],
        out_shape=[
            jax.ShapeDtypeStruct((N_NODES, D), jnp.float32),
            jax.ShapeDtypeStruct((N_NODES, D), jnp.float32),
        ],
    )(x, wlT, wrT, b1)


def _tc2_body(s0_ref, s1_ref, c0_ref, c1_ref, z_ref, w2l_ref, w2r_ref,
              h_ref, s_ref, t_ref):
    cnt = jnp.maximum(c0_ref[...] + c1_ref[...], 1.0)
    hb = (s0_ref[...] + s1_ref[...]) / cnt + z_ref[...]
    h_ref[...] = hb
    r = jnp.maximum(hb, 0.0)
    s_ref[...] = jnp.sum(r * w2l_ref[...], axis=1, keepdims=True)
    t_ref[...] = jnp.sum(r * w2r_ref[...], axis=1, keepdims=True)


def _tc2(s0, s1, c0, c1, z, w2l, w2r):
    return pl.pallas_call(
        _tc2_body,
        grid=(N_NODES // _RB,),
        in_specs=[
            pl.BlockSpec((_RB, D), lambda i: (i, 0)),
            pl.BlockSpec((_RB, D), lambda i: (i, 0)),
            pl.BlockSpec((_RB, 1), lambda i: (i, 0)),
            pl.BlockSpec((_RB, 1), lambda i: (i, 0)),
            pl.BlockSpec((_RB, D), lambda i: (i, 0)),
            pl.BlockSpec((1, D), lambda i: (0, 0)),
            pl.BlockSpec((1, D), lambda i: (0, 0)),
        ],
        out_specs=[
            pl.BlockSpec((_RB, D), lambda i: (i, 0)),
            pl.BlockSpec((_RB, 1), lambda i: (i, 0)),
            pl.BlockSpec((_RB, 1), lambda i: (i, 0)),
        ],
        out_shape=[
            jax.ShapeDtypeStruct((N_NODES, D), jnp.float32),
            jax.ShapeDtypeStruct((N_NODES, 1), jnp.float32),
            jax.ShapeDtypeStruct((N_NODES, 1), jnp.float32),
        ],
    )(s0, s1, c0, c1, z, w2l, w2r)


def _tc3_body(a0_ref, a1_ref, c0_ref, c1_ref, t_ref, b2_ref, o_ref):
    cnt = jnp.maximum(c0_ref[...] + c1_ref[...], 1.0)
    val = (a0_ref[...] + a1_ref[...]) / cnt + b2_ref[0, 0] + t_ref[...]
    o_ref[...] = jax.nn.sigmoid(val)


def _tc3(a0, a1, c0, c1, tpad, b2):
    return pl.pallas_call(
        _tc3_body,
        grid=(1,),
        in_specs=[pl.BlockSpec((NPAD // D, D), lambda i: (0, 0))] * 5
        + [pl.BlockSpec((1, 1), lambda i: (0, 0), memory_space=pltpu.SMEM)],
        out_specs=pl.BlockSpec((NPAD // D, D), lambda i: (0, 0)),
        out_shape=jax.ShapeDtypeStruct((NPAD // D, D), jnp.float32),
    )(a0, a1, c0, c1, tpad, b2)


def kernel(x, edge_index, W1_l, b1_l, W1_r, W2_l, b2_l, W2_r):
    src = edge_index[0].astype(jnp.int32)
    dst = edge_index[1].astype(jnp.int32)

    zrow = jnp.zeros((ROWS_PT, D), jnp.float32)
    zcnt = jnp.zeros((CNT_PT,), jnp.float32)

    y, z = _tc1(x, W1_l.T, W1_r.T, b1_l.reshape(1, D))

    sums, cnt = _sc1(y, src, dst, zrow, zcnt)
    cnt2 = cnt.reshape(NC, NPAD)
    c0 = cnt2[0, :N_NODES].reshape(N_NODES, 1)
    c1 = cnt2[1, :N_NODES].reshape(N_NODES, 1)

    h, s, t = _tc2(sums[:N_NODES], sums[N_NODES:], c0, c1, z,
                   W2_l.reshape(1, D), W2_r.reshape(1, D))

    s_pad = jnp.pad(s.reshape(N_NODES), (0, NPAD - N_NODES))
    (sum2,) = _sc2(s_pad, src, dst, zcnt)
    sum22 = sum2.reshape(NC, NPAD // D, D)

    t_pad = jnp.pad(t.reshape(N_NODES), (0, NPAD - N_NODES))
    cpad = cnt2.reshape(NC, NPAD // D, D)
    o = _tc3(sum22[0], sum22[1], cpad[0], cpad[1],
             t_pad.reshape(NPAD // D, D), b2_l.reshape(1, 1))

    out = o.reshape(NPAD)[:N_NODES].reshape(N_NODES, 1)
    return (out, h)


# speedup vs baseline: 8.7894x; 8.7894x over previous
import functools

import jax
import jax.numpy as jnp
from jax import lax
from jax.experimental import pallas as pl
from jax.experimental.pallas import tpu as pltpu
from jax.experimental.pallas import tpu_sc as plsc

N_NODES = 10000
N_EDGES = 320000
D = 128

NC = 2
NS = 16
NW = NC * NS
CH = 128
NCHUNK = N_EDGES // CH
JMAX = (NCHUNK + NW - 1) // NW
NPAD = 10240
ROWS_PT = NPAD // NS
CNT_PT = NPAD // NS

_MESH = plsc.VectorSubcoreMesh(core_axis_name="c", subcore_axis_name="s")


def _sc1_body(y_h, src_h, dst_h, zrow_h, zcnt_h,
              sums_h, cnt_h,
              acc_s, cnt_s,
              sidx_v, didx_v, rows_v, ones_v, sem):
    cid = lax.axis_index("c")
    sid = lax.axis_index("s")
    wid = sid * NC + cid

    for k in range(CH // 16):
        ones_v[pl.ds(k * 16, 16)] = jnp.ones((16,), jnp.float32)

    pltpu.sync_copy(zrow_h, acc_s.at[pl.ds(sid * ROWS_PT, ROWS_PT)])
    pltpu.sync_copy(zcnt_h, cnt_s.at[pl.ds(sid * CNT_PT, CNT_PT)])
    plsc.subcore_barrier()

    def body(j, carry):
        c = j * NW + wid

        @pl.when(c < NCHUNK)
        def _():
            base = c * CH
            pltpu.sync_copy(src_h.at[pl.ds(base, CH)], sidx_v)
            pltpu.async_copy(y_h.at[sidx_v], rows_v, sem).wait()
            pltpu.sync_copy(dst_h.at[pl.ds(base, CH)], didx_v)
            pltpu.sync_copy(rows_v, acc_s.at[didx_v], add=True)
            pltpu.sync_copy(ones_v, cnt_s.at[didx_v], add=True)

        return carry

    lax.fori_loop(0, JMAX, body, 0)
    plsc.subcore_barrier()

    pltpu.sync_copy(
        acc_s.at[pl.ds(sid * ROWS_PT, ROWS_PT)],
        sums_h.at[pl.ds(cid * NPAD + sid * ROWS_PT, ROWS_PT)])
    pltpu.sync_copy(
        cnt_s.at[pl.ds(sid * CNT_PT, CNT_PT)],
        cnt_h.at[pl.ds(cid * NPAD + sid * CNT_PT, CNT_PT)])


_sc1 = functools.partial(
    pl.kernel,
    mesh=_MESH,
    out_type=[
        jax.ShapeDtypeStruct((NC * NPAD, D), jnp.float32),
        jax.ShapeDtypeStruct((NC * NPAD,), jnp.float32),
    ],
    scratch_types=[
        pltpu.VMEM_SHARED((NPAD, D), jnp.float32),
        pltpu.VMEM_SHARED((NPAD,), jnp.float32),
        pltpu.VMEM((CH,), jnp.int32),
        pltpu.VMEM((CH,), jnp.int32),
        pltpu.VMEM((CH, D), jnp.float32),
        pltpu.VMEM((CH,), jnp.float32),
        pltpu.SemaphoreType.DMA,
    ],
)(_sc1_body)


def _sc2_body(s_h, src_h, dst_h, zcnt_h,
              out_h,
              acc_s,
              s_v, sidx_v, didx_v, vals_v):
    cid = lax.axis_index("c")
    sid = lax.axis_index("s")
    wid = sid * NC + cid

    pltpu.sync_copy(s_h, s_v)
    pltpu.sync_copy(zcnt_h, acc_s.at[pl.ds(sid * CNT_PT, CNT_PT)])
    plsc.subcore_barrier()

    def body(j, carry):
        c = j * NW + wid

        @pl.when(c < NCHUNK)
        def _():
            base = c * CH
            pltpu.sync_copy(src_h.at[pl.ds(base, CH)], sidx_v)
            for k in range(CH // 16):
                idx16 = sidx_v[pl.ds(k * 16, 16)]
                vals_v[pl.ds(k * 16, 16)] = plsc.load_gather(s_v, [idx16])
            pltpu.sync_copy(dst_h.at[pl.ds(base, CH)], didx_v)
            pltpu.sync_copy(vals_v, acc_s.at[didx_v], add=True)

        return carry

    lax.fori_loop(0, JMAX, body, 0)
    plsc.subcore_barrier()

    pltpu.sync_copy(
        acc_s.at[pl.ds(sid * CNT_PT, CNT_PT)],
        out_h.at[pl.ds(cid * NPAD + sid * CNT_PT, CNT_PT)])


_sc2 = functools.partial(
    pl.kernel,
    mesh=_MESH,
    out_type=[jax.ShapeDtypeStruct((NC * NPAD,), jnp.float32)],
    compiler_params=pltpu.CompilerParams(needs_layout_passes=False),
    scratch_types=[
        pltpu.VMEM_SHARED((NPAD,), jnp.float32),
        pltpu.VMEM((NPAD,), jnp.float32),
        pltpu.VMEM((CH,), jnp.int32),
        pltpu.VMEM((CH,), jnp.int32),
        pltpu.VMEM((CH,), jnp.float32),
    ],
)(_sc2_body)


_RB = 1000


def _tc1_body(x_ref, wl_ref, wr_ref, b1_ref, y_ref, z_ref):
    xb = x_ref[...]
    y_ref[...] = jnp.dot(xb, wl_ref[...], preferred_element_type=jnp.float32)
    z_ref[...] = (jnp.dot(xb, wr_ref[...], preferred_element_type=jnp.float32)
                  + b1_ref[...])


def _tc1(x, wlT, wrT, b1):
    return pl.pallas_call(
        _tc1_body,
        grid=(N_NODES // _RB,),
        in_specs=[
            pl.BlockSpec((_RB, D), lambda i: (i, 0)),
            pl.BlockSpec((D, D), lambda i: (0, 0)),
            pl.BlockSpec((D, D), lambda i: (0, 0)),
            pl.BlockSpec((1, D), lambda i: (0, 0)),
        ],
        out_specs=[
            pl.BlockSpec((_RB, D), lambda i: (i, 0)),
            pl.BlockSpec((_RB, D), lambda i: (i, 0)),
        ],
        out_shape=[
            jax.ShapeDtypeStruct((N_NODES, D), jnp.float32),
            jax.ShapeDtypeStruct((N_NODES, D), jnp.float32),
        ],
    )(x, wlT, wrT, b1)


def _tc2_body(s0_ref, s1_ref, c0_ref, c1_ref, z_ref, w2l_ref, w2r_ref,
              h_ref, s_ref, t_ref):
    cnt = jnp.maximum(c0_ref[...] + c1_ref[...], 1.0)
    hb = (s0_ref[...] + s1_ref[...]) / cnt + z_ref[...]
    h_ref[...] = hb
    r = jnp.maximum(hb, 0.0)
    s_ref[...] = jnp.sum(r * w2l_ref[...], axis=1, keepdims=True)
    t_ref[...] = jnp.sum(r * w2r_ref[...], axis=1, keepdims=True)


def _tc2(s0, s1, c0, c1, z, w2l, w2r):
    return pl.pallas_call(
        _tc2_body,
        grid=(N_NODES // _RB,),
        in_specs=[
            pl.BlockSpec((_RB, D), lambda i: (i, 0)),
            pl.BlockSpec((_RB, D), lambda i: (i, 0)),
            pl.BlockSpec((_RB, 1), lambda i: (i, 0)),
            pl.BlockSpec((_RB, 1), lambda i: (i, 0)),
            pl.BlockSpec((_RB, D), lambda i: (i, 0)),
            pl.BlockSpec((1, D), lambda i: (0, 0)),
            pl.BlockSpec((1, D), lambda i: (0, 0)),
        ],
        out_specs=[
            pl.BlockSpec((_RB, D), lambda i: (i, 0)),
            pl.BlockSpec((_RB, 1), lambda i: (i, 0)),
            pl.BlockSpec((_RB, 1), lambda i: (i, 0)),
        ],
        out_shape=[
            jax.ShapeDtypeStruct((N_NODES, D), jnp.float32),
            jax.ShapeDtypeStruct((N_NODES, 1), jnp.float32),
            jax.ShapeDtypeStruct((N_NODES, 1), jnp.float32),
        ],
    )(s0, s1, c0, c1, z, w2l, w2r)


def _tc3_body(a0_ref, a1_ref, c0_ref, c1_ref, t_ref, b2_ref, o_ref):
    cnt = jnp.maximum(c0_ref[...] + c1_ref[...], 1.0)
    val = (a0_ref[...] + a1_ref[...]) / cnt + b2_ref[0, 0] + t_ref[...]
    o_ref[...] = jax.nn.sigmoid(val)


def _tc3(a0, a1, c0, c1, tpad, b2):
    return pl.pallas_call(
        _tc3_body,
        grid=(1,),
        in_specs=[pl.BlockSpec((NPAD // D, D), lambda i: (0, 0))] * 5
        + [pl.BlockSpec((1, 1), lambda i: (0, 0), memory_space=pltpu.SMEM)],
        out_specs=pl.BlockSpec((NPAD // D, D), lambda i: (0, 0)),
        out_shape=jax.ShapeDtypeStruct((NPAD // D, D), jnp.float32),
    )(a0, a1, c0, c1, tpad, b2)


def kernel(x, edge_index, W1_l, b1_l, W1_r, W2_l, b2_l, W2_r):
    src = edge_index[0].astype(jnp.int32)
    dst = edge_index[1].astype(jnp.int32)

    zrow = jnp.zeros((ROWS_PT, D), jnp.float32)
    zcnt = jnp.zeros((CNT_PT,), jnp.float32)

    y, z = _tc1(x, W1_l.T, W1_r.T, b1_l.reshape(1, D))

    sums, cnt = _sc1(y, src, dst, zrow, zcnt)
    cnt2 = cnt.reshape(NC, NPAD)
    c0 = cnt2[0, :N_NODES].reshape(N_NODES, 1)
    c1 = cnt2[1, :N_NODES].reshape(N_NODES, 1)

    h, s, t = _tc2(sums[:N_NODES], sums[NPAD:NPAD + N_NODES], c0, c1, z,
                   W2_l.reshape(1, D), W2_r.reshape(1, D))

    s_pad = jnp.pad(s.reshape(N_NODES), (0, NPAD - N_NODES))
    (sum2,) = _sc2(s_pad, src, dst, zcnt)
    sum22 = sum2.reshape(NC, NPAD // D, D)

    t_pad = jnp.pad(t.reshape(N_NODES), (0, NPAD - N_NODES))
    cpad = cnt2.reshape(NC, NPAD // D, D)
    o = _tc3(sum22[0], sum22[1], cpad[0], cpad[1],
             t_pad.reshape(NPAD // D, D), b2_l.reshape(1, 1))

    out = o.reshape(NPAD)[:N_NODES].reshape(N_NODES, 1)
    return (out, h)


# speedup vs baseline: 16.3072x; 1.8553x over previous
import functools

import jax
import jax.numpy as jnp
from jax import lax
from jax.experimental import pallas as pl
from jax.experimental.pallas import tpu as pltpu
from jax.experimental.pallas import tpu_sc as plsc

N_NODES = 10000
N_EDGES = 320000
D = 128

NC = 2
NS = 16
NW = NC * NS
CH = 128
NCHUNK = N_EDGES // CH
JMAX = (NCHUNK + NW - 1) // NW
NPAD = 10240
ROWS_PT = NPAD // NS
CNT_PT = NPAD // NS

_MESH = plsc.VectorSubcoreMesh(core_axis_name="c", subcore_axis_name="s")


NBUF = 2
NOUT1 = 40


def _sc1_body(y_h, src_h, dst_h, zrow_h, zcnt_h,
              sums_h, cnt_h,
              acc_s, cnt_s,
              sidx_v, didx_v, rows_v, ones_v,
              si_sem, di_sem, g_sem):
    cid = lax.axis_index("c")
    sid = lax.axis_index("s")
    wid = sid * NC + cid

    for k in range(CH // 16):
        ones_v[pl.ds(k * 16, 16)] = jnp.ones((16,), jnp.float32)

    pltpu.sync_copy(zrow_h, acc_s.at[pl.ds(sid * ROWS_PT, ROWS_PT)])
    pltpu.sync_copy(zcnt_h, cnt_s.at[pl.ds(sid * CNT_PT, CNT_PT)])
    plsc.subcore_barrier()

    def chunk_of(g):
        return g * NW + wid

    def launch_idx(g, b):
        @pl.when(chunk_of(g) < NCHUNK)
        def _():
            base = chunk_of(g) * CH
            pltpu.async_copy(src_h.at[pl.ds(base, CH)], sidx_v.at[b],
                             si_sem.at[b])
            pltpu.async_copy(dst_h.at[pl.ds(base, CH)], didx_v.at[b],
                             di_sem.at[b])

    launch_idx(0, 0)

    def outer(j, carry):
        for b in range(NBUF):
            g = j * NBUF + b
            bprev = 1 - b

            @pl.when(chunk_of(g) < NCHUNK)
            def _():
                pltpu.make_async_copy(src_h.at[pl.ds(0, CH)], sidx_v.at[b],
                                      si_sem.at[b]).wait()
                pltpu.async_copy(y_h.at[sidx_v.at[b]], rows_v.at[b],
                                 g_sem.at[b])

            @pl.when((g >= 1) & (chunk_of(g - 1) < NCHUNK))
            def _():
                pltpu.make_async_copy(y_h.at[sidx_v.at[bprev]],
                                      rows_v.at[bprev],
                                      g_sem.at[bprev]).wait()
                pltpu.make_async_copy(dst_h.at[pl.ds(0, CH)],
                                      didx_v.at[bprev],
                                      di_sem.at[bprev]).wait()
                pltpu.sync_copy(rows_v.at[bprev], acc_s.at[didx_v.at[bprev]],
                                add=True)
                pltpu.sync_copy(ones_v, cnt_s.at[didx_v.at[bprev]], add=True)

            launch_idx(g + 1, bprev)
        return carry

    lax.fori_loop(0, NOUT1, outer, 0)
    plsc.subcore_barrier()

    pltpu.sync_copy(
        acc_s.at[pl.ds(sid * ROWS_PT, ROWS_PT)],
        sums_h.at[pl.ds(cid * NPAD + sid * ROWS_PT, ROWS_PT)])
    pltpu.sync_copy(
        cnt_s.at[pl.ds(sid * CNT_PT, CNT_PT)],
        cnt_h.at[pl.ds(cid * NPAD + sid * CNT_PT, CNT_PT)])


_sc1 = functools.partial(
    pl.kernel,
    mesh=_MESH,
    out_type=[
        jax.ShapeDtypeStruct((NC * NPAD, D), jnp.float32),
        jax.ShapeDtypeStruct((NC * NPAD,), jnp.float32),
    ],
    scratch_types=[
        pltpu.VMEM_SHARED((NPAD, D), jnp.float32),
        pltpu.VMEM_SHARED((NPAD,), jnp.float32),
        pltpu.VMEM((NBUF, CH), jnp.int32),
        pltpu.VMEM((NBUF, CH), jnp.int32),
        pltpu.VMEM((NBUF, CH, D), jnp.float32),
        pltpu.VMEM((CH,), jnp.float32),
        pltpu.SemaphoreType.DMA((NBUF,)),
        pltpu.SemaphoreType.DMA((NBUF,)),
        pltpu.SemaphoreType.DMA((NBUF,)),
    ],
)(_sc1_body)


G2 = 512
NG2 = N_EDGES // G2
NOUT2 = 10


def _sc2_body(s_h, src_h, dst_h, zcnt_h,
              out_h,
              acc_s,
              s_v, sidx_v, didx_v, vals_v,
              si_sem, di_sem):
    cid = lax.axis_index("c")
    sid = lax.axis_index("s")
    wid = sid * NC + cid

    pltpu.sync_copy(s_h, s_v)
    pltpu.sync_copy(zcnt_h, acc_s.at[pl.ds(sid * CNT_PT, CNT_PT)])
    plsc.subcore_barrier()

    def group_of(g):
        return g * NW + wid

    def launch_idx(g, b):
        @pl.when(group_of(g) < NG2)
        def _():
            base = group_of(g) * G2
            pltpu.async_copy(src_h.at[pl.ds(base, G2)], sidx_v.at[b],
                             si_sem.at[b])
            for k in range(G2 // CH):
                pltpu.async_copy(dst_h.at[pl.ds(base + k * CH, CH)],
                                 didx_v.at[b, k], di_sem.at[b])

    launch_idx(0, 0)

    def outer(j, carry):
        for b in range(2):
            g = j * 2 + b
            launch_idx(g + 1, 1 - b)

            @pl.when(group_of(g) < NG2)
            def _():
                pltpu.make_async_copy(src_h.at[pl.ds(0, G2)], sidx_v.at[b],
                                      si_sem.at[b]).wait()
                for k in range(G2 // CH):
                    pltpu.make_async_copy(dst_h.at[pl.ds(0, CH)],
                                          didx_v.at[b, k],
                                          di_sem.at[b]).wait()
                for k2 in range(G2 // CH):
                    for k in range(CH // 16):
                        off = k2 * CH + k * 16
                        idx16 = sidx_v[b, pl.ds(off, 16)]
                        vals_v[pl.ds(k * 16, 16)] = plsc.load_gather(
                            s_v, [idx16])
                    pltpu.sync_copy(vals_v, acc_s.at[didx_v.at[b, k2]],
                                    add=True)
        return carry

    lax.fori_loop(0, NOUT2, outer, 0)
    plsc.subcore_barrier()

    pltpu.sync_copy(
        acc_s.at[pl.ds(sid * CNT_PT, CNT_PT)],
        out_h.at[pl.ds(cid * NPAD + sid * CNT_PT, CNT_PT)])


_sc2 = functools.partial(
    pl.kernel,
    mesh=_MESH,
    out_type=[jax.ShapeDtypeStruct((NC * NPAD,), jnp.float32)],
    compiler_params=pltpu.CompilerParams(needs_layout_passes=False),
    scratch_types=[
        pltpu.VMEM_SHARED((NPAD,), jnp.float32),
        pltpu.VMEM((NPAD,), jnp.float32),
        pltpu.VMEM((2, G2), jnp.int32),
        pltpu.VMEM((2, G2 // CH, CH), jnp.int32),
        pltpu.VMEM((CH,), jnp.float32),
        pltpu.SemaphoreType.DMA((2,)),
        pltpu.SemaphoreType.DMA((2,)),
    ],
)(_sc2_body)


_RB = 1000


def _tc1_body(x_ref, wl_ref, wr_ref, b1_ref, y_ref, z_ref):
    xb = x_ref[...]
    y_ref[...] = jnp.dot(xb, wl_ref[...], preferred_element_type=jnp.float32)
    z_ref[...] = (jnp.dot(xb, wr_ref[...], preferred_element_type=jnp.float32)
                  + b1_ref[...])


def _tc1(x, wlT, wrT, b1):
    return pl.pallas_call(
        _tc1_body,
        grid=(N_NODES // _RB,),
        in_specs=[
            pl.BlockSpec((_RB, D), lambda i: (i, 0)),
            pl.BlockSpec((D, D), lambda i: (0, 0)),
            pl.BlockSpec((D, D), lambda i: (0, 0)),
            pl.BlockSpec((1, D), lambda i: (0, 0)),
        ],
        out_specs=[
            pl.BlockSpec((_RB, D), lambda i: (i, 0)),
            pl.BlockSpec((_RB, D), lambda i: (i, 0)),
        ],
        out_shape=[
            jax.ShapeDtypeStruct((N_NODES, D), jnp.float32),
            jax.ShapeDtypeStruct((N_NODES, D), jnp.float32),
        ],
    )(x, wlT, wrT, b1)


def _tc2_body(s0_ref, s1_ref, c0_ref, c1_ref, z_ref, w2l_ref, w2r_ref,
              h_ref, s_ref, t_ref):
    cnt = jnp.maximum(c0_ref[...] + c1_ref[...], 1.0)
    hb = (s0_ref[...] + s1_ref[...]) / cnt + z_ref[...]
    h_ref[...] = hb
    r = jnp.maximum(hb, 0.0)
    s_ref[...] = jnp.sum(r * w2l_ref[...], axis=1, keepdims=True)
    t_ref[...] = jnp.sum(r * w2r_ref[...], axis=1, keepdims=True)


def _tc2(s0, s1, c0, c1, z, w2l, w2r):
    return pl.pallas_call(
        _tc2_body,
        grid=(N_NODES // _RB,),
        in_specs=[
            pl.BlockSpec((_RB, D), lambda i: (i, 0)),
            pl.BlockSpec((_RB, D), lambda i: (i, 0)),
            pl.BlockSpec((_RB, 1), lambda i: (i, 0)),
            pl.BlockSpec((_RB, 1), lambda i: (i, 0)),
            pl.BlockSpec((_RB, D), lambda i: (i, 0)),
            pl.BlockSpec((1, D), lambda i: (0, 0)),
            pl.BlockSpec((1, D), lambda i: (0, 0)),
        ],
        out_specs=[
            pl.BlockSpec((_RB, D), lambda i: (i, 0)),
            pl.BlockSpec((_RB, 1), lambda i: (i, 0)),
            pl.BlockSpec((_RB, 1), lambda i: (i, 0)),
        ],
        out_shape=[
            jax.ShapeDtypeStruct((N_NODES, D), jnp.float32),
            jax.ShapeDtypeStruct((N_NODES, 1), jnp.float32),
            jax.ShapeDtypeStruct((N_NODES, 1), jnp.float32),
        ],
    )(s0, s1, c0, c1, z, w2l, w2r)


def _tc3_body(a0_ref, a1_ref, c0_ref, c1_ref, t_ref, b2_ref, o_ref):
    cnt = jnp.maximum(c0_ref[...] + c1_ref[...], 1.0)
    val = (a0_ref[...] + a1_ref[...]) / cnt + b2_ref[0, 0] + t_ref[...]
    o_ref[...] = jax.nn.sigmoid(val)


def _tc3(a0, a1, c0, c1, tpad, b2):
    return pl.pallas_call(
        _tc3_body,
        grid=(1,),
        in_specs=[pl.BlockSpec((NPAD // D, D), lambda i: (0, 0))] * 5
        + [pl.BlockSpec((1, 1), lambda i: (0, 0), memory_space=pltpu.SMEM)],
        out_specs=pl.BlockSpec((NPAD // D, D), lambda i: (0, 0)),
        out_shape=jax.ShapeDtypeStruct((NPAD // D, D), jnp.float32),
    )(a0, a1, c0, c1, tpad, b2)


def kernel(x, edge_index, W1_l, b1_l, W1_r, W2_l, b2_l, W2_r):
    src = edge_index[0].astype(jnp.int32)
    dst = edge_index[1].astype(jnp.int32)

    zrow = jnp.zeros((ROWS_PT, D), jnp.float32)
    zcnt = jnp.zeros((CNT_PT,), jnp.float32)

    y, z = _tc1(x, W1_l.T, W1_r.T, b1_l.reshape(1, D))

    sums, cnt = _sc1(y, src, dst, zrow, zcnt)
    cnt2 = cnt.reshape(NC, NPAD)
    c0 = cnt2[0, :N_NODES].reshape(N_NODES, 1)
    c1 = cnt2[1, :N_NODES].reshape(N_NODES, 1)

    h, s, t = _tc2(sums[:N_NODES], sums[NPAD:NPAD + N_NODES], c0, c1, z,
                   W2_l.reshape(1, D), W2_r.reshape(1, D))

    s_pad = jnp.pad(s.reshape(N_NODES), (0, NPAD - N_NODES))
    (sum2,) = _sc2(s_pad, src, dst, zcnt)
    sum22 = sum2.reshape(NC, NPAD // D, D)

    t_pad = jnp.pad(t.reshape(N_NODES), (0, NPAD - N_NODES))
    cpad = cnt2.reshape(NC, NPAD // D, D)
    o = _tc3(sum22[0], sum22[1], cpad[0], cpad[1],
             t_pad.reshape(NPAD // D, D), b2_l.reshape(1, 1))

    out = o.reshape(NPAD)[:N_NODES].reshape(N_NODES, 1)
    return (out, h)


# speedup vs baseline: 18.1651x; 1.1139x over previous
import functools

import jax
import jax.numpy as jnp
from jax import lax
from jax.experimental import pallas as pl
from jax.experimental.pallas import tpu as pltpu
from jax.experimental.pallas import tpu_sc as plsc

N_NODES = 10000
N_EDGES = 320000
D = 128

NC = 2
NS = 16
NW = NC * NS
CH = 128
NCHUNK = N_EDGES // CH
NPAD = 10240
ROWS_PT = NPAD // NS
CNT_PT = NPAD // NS

_MESH = plsc.VectorSubcoreMesh(core_axis_name="c", subcore_axis_name="s")


NB2 = 2
NB4 = 4
NOUT1 = 21


def _sc1_body(y_h, src_h, dst_h, zrow_h, zcnt_h,
              sums_h, cnt_h,
              acc_s, cnt_s,
              sidx_v, didx_v, rows_v, ones_v,
              si_sem, di_sem, g_sem, s_sem, c_sem):
    cid = lax.axis_index("c")
    sid = lax.axis_index("s")
    wid = sid * NC + cid

    for k in range(CH // 16):
        ones_v[pl.ds(k * 16, 16)] = jnp.ones((16,), jnp.float32)

    pltpu.sync_copy(zrow_h, acc_s.at[pl.ds(sid * ROWS_PT, ROWS_PT)])
    pltpu.sync_copy(zcnt_h, cnt_s.at[pl.ds(sid * CNT_PT, CNT_PT)])
    plsc.subcore_barrier()

    def chunk_of(g):
        return g * NW + wid

    def launch_idx(g, b2, b4):
        @pl.when(chunk_of(g) < NCHUNK)
        def _():
            base = chunk_of(g) * CH
            pltpu.async_copy(src_h.at[pl.ds(base, CH)], sidx_v.at[b2],
                             si_sem.at[b2])
            pltpu.async_copy(dst_h.at[pl.ds(base, CH)], didx_v.at[b4],
                             di_sem.at[b4])

    launch_idx(0, 0, 0)

    def outer(j, carry):
        for b in range(NB4):
            g = j * NB4 + b
            c2 = b % NB2
            p2 = (b - 1) % NB2
            c4 = b
            p4 = (b - 1) % NB4
            r4 = (b - 2) % NB4
            n2 = (b + 1) % NB2
            n4 = (b + 1) % NB4

            @pl.when((g >= 2) & (chunk_of(g - 2) < NCHUNK))
            def _():
                pltpu.make_async_copy(rows_v.at[c2], acc_s.at[didx_v.at[r4]],
                                      s_sem.at[c2]).wait()
                pltpu.make_async_copy(ones_v, cnt_s.at[didx_v.at[r4]],
                                      c_sem.at[c2]).wait()

            @pl.when(chunk_of(g) < NCHUNK)
            def _():
                pltpu.make_async_copy(src_h.at[pl.ds(0, CH)], sidx_v.at[c2],
                                      si_sem.at[c2]).wait()
                pltpu.async_copy(y_h.at[sidx_v.at[c2]], rows_v.at[c2],
                                 g_sem.at[c2])

            @pl.when((g >= 1) & (chunk_of(g - 1) < NCHUNK))
            def _():
                pltpu.make_async_copy(y_h.at[sidx_v.at[p2]],
                                      rows_v.at[p2],
                                      g_sem.at[p2]).wait()
                pltpu.make_async_copy(dst_h.at[pl.ds(0, CH)],
                                      didx_v.at[p4],
                                      di_sem.at[p4]).wait()
                pltpu.async_copy(rows_v.at[p2], acc_s.at[didx_v.at[p4]],
                                 s_sem.at[p2], add=True)
                pltpu.async_copy(ones_v, cnt_s.at[didx_v.at[p4]],
                                 c_sem.at[p2], add=True)

            launch_idx(g + 1, n2, n4)
        return carry

    lax.fori_loop(0, NOUT1, outer, 0)
    plsc.subcore_barrier()

    pltpu.sync_copy(
        acc_s.at[pl.ds(sid * ROWS_PT, ROWS_PT)],
        sums_h.at[pl.ds(cid * NPAD + sid * ROWS_PT, ROWS_PT)])
    pltpu.sync_copy(
        cnt_s.at[pl.ds(sid * CNT_PT, CNT_PT)],
        cnt_h.at[pl.ds(cid * NPAD + sid * CNT_PT, CNT_PT)])


_sc1 = functools.partial(
    pl.kernel,
    mesh=_MESH,
    out_type=[
        jax.ShapeDtypeStruct((NC * NPAD, D), jnp.float32),
        jax.ShapeDtypeStruct((NC * NPAD,), jnp.float32),
    ],
    scratch_types=[
        pltpu.VMEM_SHARED((NPAD, D), jnp.float32),
        pltpu.VMEM_SHARED((NPAD,), jnp.float32),
        pltpu.VMEM((NB2, CH), jnp.int32),
        pltpu.VMEM((NB4, CH), jnp.int32),
        pltpu.VMEM((NB2, CH, D), jnp.float32),
        pltpu.VMEM((CH,), jnp.float32),
        pltpu.SemaphoreType.DMA((NB2,)),
        pltpu.SemaphoreType.DMA((NB4,)),
        pltpu.SemaphoreType.DMA((NB2,)),
        pltpu.SemaphoreType.DMA((NB2,)),
        pltpu.SemaphoreType.DMA((NB2,)),
    ],
)(_sc1_body)


G2 = 512
NG2 = N_EDGES // G2
NOUT2 = 10


def _sc2_body(s_h, src_h, dst_h, zcnt_h,
              out_h,
              acc_s,
              s_v, sidx_v, didx_v, vals_v,
              si_sem, di_sem):
    cid = lax.axis_index("c")
    sid = lax.axis_index("s")
    wid = sid * NC + cid

    pltpu.sync_copy(s_h, s_v)
    pltpu.sync_copy(zcnt_h, acc_s.at[pl.ds(sid * CNT_PT, CNT_PT)])
    plsc.subcore_barrier()

    def group_of(g):
        return g * NW + wid

    def launch_idx(g, b):
        @pl.when(group_of(g) < NG2)
        def _():
            base = group_of(g) * G2
            pltpu.async_copy(src_h.at[pl.ds(base, G2)], sidx_v.at[b],
                             si_sem.at[b])
            for k in range(G2 // CH):
                pltpu.async_copy(dst_h.at[pl.ds(base + k * CH, CH)],
                                 didx_v.at[b, k], di_sem.at[b])

    launch_idx(0, 0)

    def outer(j, carry):
        for b in range(2):
            g = j * 2 + b
            launch_idx(g + 1, 1 - b)

            @pl.when(group_of(g) < NG2)
            def _():
                pltpu.make_async_copy(src_h.at[pl.ds(0, G2)], sidx_v.at[b],
                                      si_sem.at[b]).wait()
                for k in range(G2 // CH):
                    pltpu.make_async_copy(dst_h.at[pl.ds(0, CH)],
                                          didx_v.at[b, k],
                                          di_sem.at[b]).wait()
                for k2 in range(G2 // CH):
                    for k in range(CH // 16):
                        off = k2 * CH + k * 16
                        idx16 = sidx_v[b, pl.ds(off, 16)]
                        vals_v[pl.ds(k * 16, 16)] = plsc.load_gather(
                            s_v, [idx16])
                    pltpu.sync_copy(vals_v, acc_s.at[didx_v.at[b, k2]],
                                    add=True)
        return carry

    lax.fori_loop(0, NOUT2, outer, 0)
    plsc.subcore_barrier()

    pltpu.sync_copy(
        acc_s.at[pl.ds(sid * CNT_PT, CNT_PT)],
        out_h.at[pl.ds(cid * NPAD + sid * CNT_PT, CNT_PT)])


_sc2 = functools.partial(
    pl.kernel,
    mesh=_MESH,
    out_type=[jax.ShapeDtypeStruct((NC * NPAD,), jnp.float32)],
    compiler_params=pltpu.CompilerParams(needs_layout_passes=False),
    scratch_types=[
        pltpu.VMEM_SHARED((NPAD,), jnp.float32),
        pltpu.VMEM((NPAD,), jnp.float32),
        pltpu.VMEM((2, G2), jnp.int32),
        pltpu.VMEM((2, G2 // CH, CH), jnp.int32),
        pltpu.VMEM((CH,), jnp.float32),
        pltpu.SemaphoreType.DMA((2,)),
        pltpu.SemaphoreType.DMA((2,)),
    ],
)(_sc2_body)


_RB1 = 1000
_RB2 = 1024


def _tc1_body(x_ref, wl_ref, wr_ref, b1_ref, y_ref, z_ref):
    xb = x_ref[...]
    y_ref[...] = jnp.dot(xb, wl_ref[...], preferred_element_type=jnp.float32)
    z_ref[...] = (jnp.dot(xb, wr_ref[...], preferred_element_type=jnp.float32)
                  + b1_ref[...])


def _tc1(x, wlT, wrT, b1):
    return pl.pallas_call(
        _tc1_body,
        grid=(N_NODES // _RB1,),
        in_specs=[
            pl.BlockSpec((_RB1, D), lambda i: (i, 0)),
            pl.BlockSpec((D, D), lambda i: (0, 0)),
            pl.BlockSpec((D, D), lambda i: (0, 0)),
            pl.BlockSpec((1, D), lambda i: (0, 0)),
        ],
        out_specs=[
            pl.BlockSpec((_RB1, D), lambda i: (i, 0)),
            pl.BlockSpec((_RB1, D), lambda i: (i, 0)),
        ],
        out_shape=[
            jax.ShapeDtypeStruct((NPAD, D), jnp.float32),
            jax.ShapeDtypeStruct((NPAD, D), jnp.float32),
        ],
    )(x, wlT, wrT, b1)


def _tc2_body(s0_ref, s1_ref, c0_ref, c1_ref, z_ref, w2l_ref, w2r_ref,
              h_ref, s_ref, t_ref):
    cnt = jnp.maximum(c0_ref[...] + c1_ref[...], 1.0)
    hb = (s0_ref[...] + s1_ref[...]) / cnt + z_ref[...]
    h_ref[...] = hb
    r = jnp.maximum(hb, 0.0)
    s_ref[...] = jnp.sum(r * w2l_ref[...], axis=1, keepdims=True)
    t_ref[...] = jnp.sum(r * w2r_ref[...], axis=1, keepdims=True)


def _tc2(sums, cnt2d, z, w2l, w2r):
    off = NPAD // _RB2
    return pl.pallas_call(
        _tc2_body,
        grid=(NPAD // _RB2,),
        in_specs=[
            pl.BlockSpec((_RB2, D), lambda i: (i, 0)),
            pl.BlockSpec((_RB2, D), lambda i, o=off: (i + o, 0)),
            pl.BlockSpec((_RB2, 1), lambda i: (i, 0)),
            pl.BlockSpec((_RB2, 1), lambda i, o=off: (i + o, 0)),
            pl.BlockSpec((_RB2, D), lambda i: (i, 0)),
            pl.BlockSpec((1, D), lambda i: (0, 0)),
            pl.BlockSpec((1, D), lambda i: (0, 0)),
        ],
        out_specs=[
            pl.BlockSpec((_RB2, D), lambda i: (i, 0)),
            pl.BlockSpec((_RB2, 1), lambda i: (i, 0)),
            pl.BlockSpec((_RB2, 1), lambda i: (i, 0)),
        ],
        out_shape=[
            jax.ShapeDtypeStruct((NPAD, D), jnp.float32),
            jax.ShapeDtypeStruct((NPAD, 1), jnp.float32),
            jax.ShapeDtypeStruct((NPAD, 1), jnp.float32),
        ],
    )(sums, sums, cnt2d, cnt2d, z, w2l, w2r)


def _tc3_body(a0_ref, a1_ref, c0_ref, c1_ref, t_ref, b2_ref, o_ref):
    cnt = jnp.maximum(c0_ref[...] + c1_ref[...], 1.0)
    val = (a0_ref[...] + a1_ref[...]) / cnt + b2_ref[0, 0] + t_ref[...]
    o_ref[...] = jax.nn.sigmoid(val)


def _tc3(a0, a1, c0, c1, tpad, b2):
    return pl.pallas_call(
        _tc3_body,
        grid=(1,),
        in_specs=[pl.BlockSpec((NPAD // D, D), lambda i: (0, 0))] * 5
        + [pl.BlockSpec((1, 1), lambda i: (0, 0), memory_space=pltpu.SMEM)],
        out_specs=pl.BlockSpec((NPAD // D, D), lambda i: (0, 0)),
        out_shape=jax.ShapeDtypeStruct((NPAD // D, D), jnp.float32),
    )(a0, a1, c0, c1, tpad, b2)


def kernel(x, edge_index, W1_l, b1_l, W1_r, W2_l, b2_l, W2_r):
    src = edge_index[0].astype(jnp.int32)
    dst = edge_index[1].astype(jnp.int32)

    zrow = jnp.zeros((ROWS_PT, D), jnp.float32)
    zcnt = jnp.zeros((CNT_PT,), jnp.float32)

    y, z = _tc1(x, W1_l.T, W1_r.T, b1_l.reshape(1, D))

    sums, cnt = _sc1(y, src, dst, zrow, zcnt)

    h, s, t = _tc2(sums, cnt.reshape(NC * NPAD, 1), z,
                   W2_l.reshape(1, D), W2_r.reshape(1, D))

    (sum2,) = _sc2(s.reshape(NPAD), src, dst, zcnt)

    sum22 = sum2.reshape(NC, NPAD // D, D)
    cpad = cnt.reshape(NC, NPAD // D, D)
    o = _tc3(sum22[0], sum22[1], cpad[0], cpad[1],
             t.reshape(NPAD // D, D), b2_l.reshape(1, 1))

    out = o.reshape(NPAD)[:N_NODES].reshape(N_NODES, 1)
    return (out, h[:N_NODES])


# speedup vs baseline: 18.5335x; 1.0203x over previous
import functools

import jax
import jax.numpy as jnp
from jax import lax
from jax.experimental import pallas as pl
from jax.experimental.pallas import tpu as pltpu
from jax.experimental.pallas import tpu_sc as plsc

N_NODES = 10000
N_EDGES = 320000
D = 128

NC = 2
NS = 16
NW = NC * NS
CH = 128
NCHUNK = N_EDGES // CH
NPAD = 10240
ROWS_PT = NPAD // NS
CNT_PT = NPAD // NS

_MESH = plsc.VectorSubcoreMesh(core_axis_name="c", subcore_axis_name="s")


NB2 = 2
NB4 = 4
NOUT1 = 21


def _sc1_body(y_h, src_h, dst_h, zrow_h, zcnt_h,
              sums_h, cnt_h,
              acc_s,
              sidx_v, didx_v, rows_v, cntp_v,
              si_sem, di_sem, g_sem, s_sem):
    cid = lax.axis_index("c")
    sid = lax.axis_index("s")
    wid = sid * NC + cid
    ones16 = jnp.ones((16,), jnp.float32)

    pltpu.sync_copy(zrow_h, acc_s.at[pl.ds(sid * ROWS_PT, ROWS_PT)])
    pltpu.sync_copy(zcnt_h, cntp_v)
    plsc.subcore_barrier()

    def chunk_of(g):
        return g * NW + wid

    def launch_idx(g, b2, b4):
        @pl.when(chunk_of(g) < NCHUNK)
        def _():
            base = chunk_of(g) * CH
            pltpu.async_copy(src_h.at[pl.ds(base, CH)], sidx_v.at[b2],
                             si_sem.at[b2])
            pltpu.async_copy(dst_h.at[pl.ds(base, CH)], didx_v.at[b4],
                             di_sem.at[b4])

    launch_idx(0, 0, 0)

    def outer(j, carry):
        for b in range(NB4):
            g = j * NB4 + b
            c2 = b % NB2
            p2 = (b - 1) % NB2
            p4 = (b - 1) % NB4
            r4 = (b - 2) % NB4
            n2 = (b + 1) % NB2
            n4 = (b + 1) % NB4

            @pl.when((g >= 2) & (chunk_of(g - 2) < NCHUNK))
            def _():
                pltpu.make_async_copy(rows_v.at[c2], acc_s.at[didx_v.at[r4]],
                                      s_sem.at[c2]).wait()

            @pl.when(chunk_of(g) < NCHUNK)
            def _():
                pltpu.make_async_copy(src_h.at[pl.ds(0, CH)], sidx_v.at[c2],
                                      si_sem.at[c2]).wait()
                pltpu.async_copy(y_h.at[sidx_v.at[c2]], rows_v.at[c2],
                                 g_sem.at[c2])

            @pl.when((g >= 1) & (chunk_of(g - 1) < NCHUNK))
            def _():
                pltpu.make_async_copy(y_h.at[sidx_v.at[p2]],
                                      rows_v.at[p2],
                                      g_sem.at[p2]).wait()
                pltpu.make_async_copy(dst_h.at[pl.ds(0, CH)],
                                      didx_v.at[p4],
                                      di_sem.at[p4]).wait()
                pltpu.async_copy(rows_v.at[p2], acc_s.at[didx_v.at[p4]],
                                 s_sem.at[p2], add=True)
                for k in range(CH // 16):
                    idx16 = didx_v[p4, pl.ds(k * 16, 16)]
                    plsc.addupdate_scatter(cntp_v, [idx16], ones16)

            launch_idx(g + 1, n2, n4)
        return carry

    lax.fori_loop(0, NOUT1, outer, 0)
    plsc.subcore_barrier()

    pltpu.sync_copy(
        acc_s.at[pl.ds(sid * ROWS_PT, ROWS_PT)],
        sums_h.at[pl.ds(cid * NPAD + sid * ROWS_PT, ROWS_PT)])
    pltpu.sync_copy(cntp_v, cnt_h.at[pl.ds(wid * NPAD, NPAD)])


_sc1 = functools.partial(
    pl.kernel,
    mesh=_MESH,
    out_type=[
        jax.ShapeDtypeStruct((NC * NPAD, D), jnp.float32),
        jax.ShapeDtypeStruct((NW * NPAD,), jnp.float32),
    ],
    compiler_params=pltpu.CompilerParams(needs_layout_passes=False),
    scratch_types=[
        pltpu.VMEM_SHARED((NPAD, D), jnp.float32),
        pltpu.VMEM((NB2, CH), jnp.int32),
        pltpu.VMEM((NB4, CH), jnp.int32),
        pltpu.VMEM((NB2, CH, D), jnp.float32),
        pltpu.VMEM((NPAD,), jnp.float32),
        pltpu.SemaphoreType.DMA((NB2,)),
        pltpu.SemaphoreType.DMA((NB4,)),
        pltpu.SemaphoreType.DMA((NB2,)),
        pltpu.SemaphoreType.DMA((NB2,)),
    ],
)(_sc1_body)


G2 = 512
NG2 = N_EDGES // G2
NOUT2 = 10


def _sc2_body(s_h, src_h, dst_h, zcnt_h,
              out_h,
              acc_s,
              s_v, sidx_v, didx_v, vals_v,
              si_sem, di_sem):
    cid = lax.axis_index("c")
    sid = lax.axis_index("s")
    wid = sid * NC + cid

    pltpu.sync_copy(s_h, s_v)
    pltpu.sync_copy(zcnt_h, acc_s.at[pl.ds(sid * CNT_PT, CNT_PT)])
    plsc.subcore_barrier()

    def group_of(g):
        return g * NW + wid

    def launch_idx(g, b):
        @pl.when(group_of(g) < NG2)
        def _():
            base = group_of(g) * G2
            pltpu.async_copy(src_h.at[pl.ds(base, G2)], sidx_v.at[b],
                             si_sem.at[b])
            for k in range(G2 // CH):
                pltpu.async_copy(dst_h.at[pl.ds(base + k * CH, CH)],
                                 didx_v.at[b, k], di_sem.at[b])

    launch_idx(0, 0)

    def outer(j, carry):
        for b in range(2):
            g = j * 2 + b
            launch_idx(g + 1, 1 - b)

            @pl.when(group_of(g) < NG2)
            def _():
                pltpu.make_async_copy(src_h.at[pl.ds(0, G2)], sidx_v.at[b],
                                      si_sem.at[b]).wait()
                for k in range(G2 // CH):
                    pltpu.make_async_copy(dst_h.at[pl.ds(0, CH)],
                                          didx_v.at[b, k],
                                          di_sem.at[b]).wait()
                for k2 in range(G2 // CH):
                    for k in range(CH // 16):
                        off = k2 * CH + k * 16
                        idx16 = sidx_v[b, pl.ds(off, 16)]
                        vals_v[pl.ds(k * 16, 16)] = plsc.load_gather(
                            s_v, [idx16])
                    pltpu.sync_copy(vals_v, acc_s.at[didx_v.at[b, k2]],
                                    add=True)
        return carry

    lax.fori_loop(0, NOUT2, outer, 0)
    plsc.subcore_barrier()

    pltpu.sync_copy(
        acc_s.at[pl.ds(sid * CNT_PT, CNT_PT)],
        out_h.at[pl.ds(cid * NPAD + sid * CNT_PT, CNT_PT)])


_sc2 = functools.partial(
    pl.kernel,
    mesh=_MESH,
    out_type=[jax.ShapeDtypeStruct((NC * NPAD,), jnp.float32)],
    compiler_params=pltpu.CompilerParams(needs_layout_passes=False),
    scratch_types=[
        pltpu.VMEM_SHARED((NPAD,), jnp.float32),
        pltpu.VMEM((NPAD,), jnp.float32),
        pltpu.VMEM((2, G2), jnp.int32),
        pltpu.VMEM((2, G2 // CH, CH), jnp.int32),
        pltpu.VMEM((CH,), jnp.float32),
        pltpu.SemaphoreType.DMA((2,)),
        pltpu.SemaphoreType.DMA((2,)),
    ],
)(_sc2_body)


_RB1 = 1000
_RB2 = 1024


def _tc1_body(x_ref, wl_ref, wr_ref, b1_ref, y_ref, z_ref):
    xb = x_ref[...]
    y_ref[...] = jnp.dot(xb, wl_ref[...], preferred_element_type=jnp.float32)
    z_ref[...] = (jnp.dot(xb, wr_ref[...], preferred_element_type=jnp.float32)
                  + b1_ref[...])


def _tc1(x, wlT, wrT, b1):
    return pl.pallas_call(
        _tc1_body,
        grid=(N_NODES // _RB1,),
        in_specs=[
            pl.BlockSpec((_RB1, D), lambda i: (i, 0)),
            pl.BlockSpec((D, D), lambda i: (0, 0)),
            pl.BlockSpec((D, D), lambda i: (0, 0)),
            pl.BlockSpec((1, D), lambda i: (0, 0)),
        ],
        out_specs=[
            pl.BlockSpec((_RB1, D), lambda i: (i, 0)),
            pl.BlockSpec((_RB1, D), lambda i: (i, 0)),
        ],
        out_shape=[
            jax.ShapeDtypeStruct((NPAD, D), jnp.float32),
            jax.ShapeDtypeStruct((NPAD, D), jnp.float32),
        ],
    )(x, wlT, wrT, b1)


def _tc2_body(s0_ref, s1_ref, cm_ref, z_ref, w2l_ref, w2r_ref,
              h_ref, s_ref, t_ref, c_ref):
    ones_w = jnp.ones((NW, 1), jnp.float32)
    craw = lax.dot_general(cm_ref[...], ones_w, (((0,), (0,)), ((), ())),
                           preferred_element_type=jnp.float32)
    cnt = jnp.maximum(craw, 1.0)
    hb = (s0_ref[...] + s1_ref[...]) / cnt + z_ref[...]
    h_ref[...] = hb
    r = jnp.maximum(hb, 0.0)
    s_ref[...] = jnp.sum(r * w2l_ref[...], axis=1, keepdims=True)
    t_ref[...] = jnp.sum(r * w2r_ref[...], axis=1, keepdims=True)
    c_ref[...] = cnt


def _tc2(sums, cntmat, z, w2l, w2r):
    off = NPAD // _RB2
    return pl.pallas_call(
        _tc2_body,
        grid=(NPAD // _RB2,),
        in_specs=[
            pl.BlockSpec((_RB2, D), lambda i: (i, 0)),
            pl.BlockSpec((_RB2, D), lambda i, o=off: (i + o, 0)),
            pl.BlockSpec((NW, _RB2), lambda i: (0, i)),
            pl.BlockSpec((_RB2, D), lambda i: (i, 0)),
            pl.BlockSpec((1, D), lambda i: (0, 0)),
            pl.BlockSpec((1, D), lambda i: (0, 0)),
        ],
        out_specs=[
            pl.BlockSpec((_RB2, D), lambda i: (i, 0)),
            pl.BlockSpec((_RB2, 1), lambda i: (i, 0)),
            pl.BlockSpec((_RB2, 1), lambda i: (i, 0)),
            pl.BlockSpec((_RB2, 1), lambda i: (i, 0)),
        ],
        out_shape=[
            jax.ShapeDtypeStruct((NPAD, D), jnp.float32),
            jax.ShapeDtypeStruct((NPAD, 1), jnp.float32),
            jax.ShapeDtypeStruct((NPAD, 1), jnp.float32),
            jax.ShapeDtypeStruct((NPAD, 1), jnp.float32),
        ],
    )(sums, sums, cntmat, z, w2l, w2r)


def _tc3_body(a0_ref, a1_ref, c_ref, t_ref, b2_ref, o_ref):
    val = ((a0_ref[...] + a1_ref[...]) / c_ref[...] + b2_ref[0, 0]
           + t_ref[...])
    o_ref[...] = jax.nn.sigmoid(val)


def _tc3(a0, a1, c, tpad, b2):
    return pl.pallas_call(
        _tc3_body,
        grid=(1,),
        in_specs=[pl.BlockSpec((NPAD // D, D), lambda i: (0, 0))] * 4
        + [pl.BlockSpec((1, 1), lambda i: (0, 0), memory_space=pltpu.SMEM)],
        out_specs=pl.BlockSpec((NPAD // D, D), lambda i: (0, 0)),
        out_shape=jax.ShapeDtypeStruct((NPAD // D, D), jnp.float32),
    )(a0, a1, c, tpad, b2)


def kernel(x, edge_index, W1_l, b1_l, W1_r, W2_l, b2_l, W2_r):
    src = edge_index[0].astype(jnp.int32)
    dst = edge_index[1].astype(jnp.int32)

    zrow = jnp.zeros((ROWS_PT, D), jnp.float32)
    zcnt = jnp.zeros((CNT_PT,), jnp.float32)
    zcnt_full = jnp.zeros((NPAD,), jnp.float32)

    y, z = _tc1(x, W1_l.T, W1_r.T, b1_l.reshape(1, D))

    sums, cntflat = _sc1(y, src, dst, zrow, zcnt_full)

    h, s, t, c = _tc2(sums, cntflat.reshape(NW, NPAD), z,
                      W2_l.reshape(1, D), W2_r.reshape(1, D))

    (sum2,) = _sc2(s.reshape(NPAD), src, dst, zcnt)

    sum22 = sum2.reshape(NC, NPAD // D, D)
    o = _tc3(sum22[0], sum22[1], c.reshape(NPAD // D, D),
             t.reshape(NPAD // D, D), b2_l.reshape(1, 1))

    out = o.reshape(NPAD)[:N_NODES].reshape(N_NODES, 1)
    return (out, h[:N_NODES])
